# Initial kernel scaffold; baseline (speedup 1.0000x reference)
import jax
import jax.numpy as jnp
from jax.experimental import pallas as pl


def kernel(h, edge_index, d, w, gate_w, gate_b):
    raise NotImplementedError("write your pallas kernel here")


# speedup vs baseline: 8.2633x; 8.2633x over previous
import functools

import jax
import jax.numpy as jnp
from jax import lax
from jax.experimental import pallas as pl
from jax.experimental.pallas import tpu as pltpu
from jax.experimental.pallas import tpu_sc as plsc

L = 16
NW = 32
CH = 128


def _gate_matvec(h, w2, b2):
    def body(h_ref, w_ref, b_ref, o_ref):
        o_ref[...] = lax.dot_general(
            w_ref[...], h_ref[...],
            dimension_numbers=(((1,), (1,)), ((), ())),
            preferred_element_type=jnp.float32) + b_ref[...]
    return pl.pallas_call(
        body,
        out_shape=jax.ShapeDtypeStruct((2, h.shape[0]), jnp.float32),
    )(h, w2, b2)


def _combine(part, n):
    _, _, d = part.shape
    br = 2000
    def body(p_ref, o_ref):
        o_ref[...] = p_ref[0] + p_ref[1]
    return pl.pallas_call(
        body,
        grid=(n // br,),
        in_specs=[pl.BlockSpec((2, br, d), lambda i: (0, i, 0))],
        out_specs=pl.BlockSpec((br, d), lambda i: (i, 0)),
        out_shape=jax.ShapeDtypeStruct((n, d), jnp.float32),
    )(part)


@functools.cache
def _make_sc(N, D, NCH, NPAD):
    nj = D // L
    z_rows_pt = NPAD // L
    mesh = plsc.VectorSubcoreMesh(core_axis_name="c", subcore_axis_name="s")

    @functools.partial(
        pl.kernel,
        out_type=jax.ShapeDtypeStruct((2, NPAD, D), jnp.float32),
        mesh=mesh,
        compiler_params=pltpu.CompilerParams(needs_layout_passes=False),
        scratch_types=[
            pltpu.VMEM((N,), jnp.float32),
            pltpu.VMEM((N,), jnp.float32),
            pltpu.VMEM((N,), jnp.float32),
            pltpu.VMEM((CH,), jnp.int32),
            pltpu.VMEM((CH,), jnp.int32),
            pltpu.VMEM((CH,), jnp.float32),
            pltpu.VMEM((CH, D), jnp.float32),
            pltpu.VMEM_SHARED((NPAD, D), jnp.float32),
            pltpu.SemaphoreType.DMA,
        ],
    )
    def sc_fn(h_hbm, a_hbm, b_hbm, d_hbm, s_hbm, t_hbm, w_hbm, z0_hbm,
              out_hbm, a_v, b_v, dd_v, s_v, t_v, w_v, rows_v, z_sh, sem):
        cid = lax.axis_index("c")
        sid = lax.axis_index("s")
        wid = sid * 2 + cid

        pltpu.sync_copy(a_hbm, a_v)
        pltpu.sync_copy(b_hbm, b_v)
        pltpu.sync_copy(d_hbm, dd_v)
        pltpu.sync_copy(z0_hbm.at[pl.ds(sid * z_rows_pt, z_rows_pt)],
                        z_sh.at[pl.ds(sid * z_rows_pt, z_rows_pt)])
        plsc.subcore_barrier()

        iota = lax.iota(jnp.int32, L)
        base_e = wid * (NCH * CH)

        def chunk(c, carry):
            off = base_e + c * CH
            pltpu.sync_copy(s_hbm.at[pl.ds(off, CH)], s_v)
            pltpu.sync_copy(t_hbm.at[pl.ds(off, CH)], t_v)
            pltpu.sync_copy(w_hbm.at[pl.ds(off, CH)], w_v)
            pltpu.async_copy(h_hbm.at[s_v], rows_v, sem).wait()

            def group(g, carry2):
                sl = pl.ds(g * L, L)
                s16 = s_v[sl]
                t16 = t_v[sl]
                w16 = w_v[sl]
                a_t = plsc.load_gather(a_v, [t16])
                b_s = plsc.load_gather(b_v, [s16])
                d_t = plsc.load_gather(dd_v, [t16])
                d_s = plsc.load_gather(dd_v, [s16])
                ex = jnp.exp((a_t + b_s) * 2.0)
                coef = (1.0 - 2.0 / (ex + 1.0)) * (d_t * d_s * w16)
                rbase = g * L + iota

                def jloop(j, carry3):
                    cbase = j * L
                    for k in range(L):
                        ccol = cbase + ((iota + k) & (L - 1))
                        v = plsc.load_gather(rows_v, [rbase, ccol])
                        plsc.store_scatter(rows_v, [rbase, ccol], v * coef)
                    return carry3

                lax.fori_loop(0, nj, jloop, 0)
                return carry2

            lax.fori_loop(0, CH // L, group, 0)
            pltpu.sync_copy(rows_v, z_sh.at[t_v], add=True)
            return carry

        lax.fori_loop(0, NCH, chunk, 0)
        plsc.subcore_barrier()
        pltpu.sync_copy(z_sh.at[pl.ds(sid * z_rows_pt, z_rows_pt)],
                        out_hbm.at[cid, pl.ds(sid * z_rows_pt, z_rows_pt)])

    return sc_fn


def kernel(h, edge_index, d, w, gate_w, gate_b):
    N, D = h.shape
    E = edge_index.shape[1]
    src = edge_index[0].astype(jnp.int32)
    dst = edge_index[1].astype(jnp.int32)

    w2 = jnp.concatenate([gate_w[:, :D], gate_w[:, D:]], axis=0)
    b2 = jnp.stack([gate_b, jnp.zeros_like(gate_b)], axis=0)
    ab = _gate_matvec(h, w2, b2)

    nch = -(-E // (NW * CH))
    e_pad = NW * CH * nch
    npad = -(-(N + 1) // 128) * 128
    pad = e_pad - E
    src_p = jnp.concatenate([src, jnp.zeros((pad,), jnp.int32)])
    dst_p = jnp.concatenate([dst, jnp.full((pad,), N, jnp.int32)])
    w_p = jnp.concatenate([w, jnp.zeros((pad,), jnp.float32)])
    z0 = jnp.zeros((npad, D), jnp.float32)

    part = _make_sc(N, D, nch, npad)(h, ab[0], ab[1], d, src_p, dst_p, w_p, z0)
    return _combine(part, N)
